# Initial kernel scaffold; baseline (speedup 1.0000x reference)
import jax
import jax.numpy as jnp
from jax.experimental import pallas as pl


def kernel(x, pos, extra_x, edge_attr, ss, time, params, edge_index, batch):
    raise NotImplementedError("write your pallas kernel here")


# speedup vs baseline: 4.5029x; 4.5029x over previous
import functools
import math

import jax
import jax.numpy as jnp
from jax import lax
from jax.experimental import pallas as pl
from jax.experimental.pallas import tpu as pltpu
from jax.experimental.pallas import tpu_sc as plsc

NND = 10000
NED = 320000
HID = 64
NL = 4
ODIM = 20
WROW = 128
NC, NS = 2, 16
NW = NC * NS
EPT = NED // NW
CH = 80
NCHUNK = EPT // CH
STRIPE = 624
STRIPE_LAST = NND - (NS - 1) * STRIPE
ET = 2000

@functools.cache
def _sc_mesh():
  return plsc.VectorSubcoreMesh(
      core_axis_name="c", subcore_axis_name="s", num_cores=NC, num_subcores=NS)


def _silu(x):
  return x * jax.nn.sigmoid(x)


def _sc_gather_body(td_hbm, ts_hbm, dst_hbm, src_hbm, outd_hbm, outs_hbm,
                    idxd_v, idxs_v, bufd0, bufd1, bufs0, bufs1,
                    gsd0, gsd1, gss0, gss1, wsd0, wsd1, wss0, wss1):
  cid = lax.axis_index("c")
  sid = lax.axis_index("s")
  wid = cid * NS + sid
  pltpu.sync_copy(dst_hbm.at[wid], idxd_v)
  pltpu.sync_copy(src_hbm.at[wid], idxs_v)
  base = wid * EPT
  bufd = (bufd0, bufd1)
  bufs = (bufs0, bufs1)
  gsd = (gsd0, gsd1)
  gss = (gss0, gss1)
  wsd = (wsd0, wsd1)
  wss = (wss0, wss1)

  def pair(j0, nb):
    gd = []
    for b in range(nb):
      j = j0 + b
      gd.append(pltpu.async_copy(td_hbm.at[idxd_v.at[j]], bufd[b], gsd[b]))
      gd.append(pltpu.async_copy(ts_hbm.at[idxs_v.at[j]], bufs[b], gss[b]))
    wr = []
    for b in range(nb):
      j = j0 + b
      row0 = base + j * CH
      gd[2 * b].wait()
      wr.append(pltpu.async_copy(bufd[b], outd_hbm.at[pl.ds(row0, CH)], wsd[b]))
      gd[2 * b + 1].wait()
      wr.append(pltpu.async_copy(bufs[b], outs_hbm.at[pl.ds(row0, CH)], wss[b]))
    for d in wr:
      d.wait()

  @pl.loop(0, NCHUNK - 1, step=2)
  def _(j0):
    pair(j0, 2)

  pair(NCHUNK - 1, 1)


@functools.cache
def _sc_gather():
  return pl.kernel(
      _sc_gather_body,
      out_type=[jax.ShapeDtypeStruct((NED, WROW), jnp.float32),
                jax.ShapeDtypeStruct((NED, WROW), jnp.float32)],
      mesh=_sc_mesh(),
      scratch_types=[pltpu.VMEM((NCHUNK, CH), jnp.int32),
                     pltpu.VMEM((NCHUNK, CH), jnp.int32)]
                    + [pltpu.VMEM((CH, WROW), jnp.float32)] * 4
                    + [pltpu.SemaphoreType.DMA] * 8,
  )


def _sc_scatter_body(pay_hbm, dst_hbm, zeros_hbm, out_hbm,
                     idx_v, pbuf0, pbuf1, sem0, sem1, acc_sh):
  cid = lax.axis_index("c")
  sid = lax.axis_index("s")
  wid = cid * NS + sid
  pltpu.sync_copy(dst_hbm.at[wid], idx_v)

  @pl.when(sid < NS - 1)
  def _():
    pltpu.sync_copy(zeros_hbm.at[pl.ds(0, STRIPE)],
                    acc_sh.at[pl.ds(sid * STRIPE, STRIPE)])

  @pl.when(sid == NS - 1)
  def _():
    pltpu.sync_copy(zeros_hbm, acc_sh.at[pl.ds((NS - 1) * STRIPE,
                                               STRIPE_LAST)])

  plsc.subcore_barrier()
  pbuf = (pbuf0, pbuf1)
  sem = (sem0, sem1)
  base = wid * EPT

  def pair(j0, nb):
    ld = []
    for b in range(nb):
      j = j0 + b
      ld.append(pltpu.async_copy(pay_hbm.at[pl.ds(base + j * CH, CH)],
                                 pbuf[b], sem[b]))
    for b in range(nb):
      ld[b].wait()
      pltpu.sync_copy(pbuf[b], acc_sh.at[idx_v.at[j0 + b]], add=True)

  @pl.loop(0, NCHUNK - 1, step=2)
  def _(j0):
    pair(j0, 2)

  pair(NCHUNK - 1, 1)

  plsc.subcore_barrier()

  @pl.when(sid < NS - 1)
  def _():
    pltpu.sync_copy(
        acc_sh.at[pl.ds(sid * STRIPE, STRIPE)],
        out_hbm.at[cid, pl.ds(sid * STRIPE, STRIPE)])

  @pl.when(sid == NS - 1)
  def _():
    pltpu.sync_copy(
        acc_sh.at[pl.ds((NS - 1) * STRIPE, STRIPE_LAST)],
        out_hbm.at[cid, pl.ds((NS - 1) * STRIPE, STRIPE_LAST)])


@functools.cache
def _sc_scatter():
  return pl.kernel(
      _sc_scatter_body,
      out_type=jax.ShapeDtypeStruct((NC, NND, WROW), jnp.float32),
      mesh=_sc_mesh(),
      scratch_types=[pltpu.VMEM((NCHUNK, CH), jnp.int32),
                     pltpu.VMEM((CH, WROW), jnp.float32),
                     pltpu.VMEM((CH, WROW), jnp.float32),
                     pltpu.SemaphoreType.DMA, pltpu.SemaphoreType.DMA,
                     pltpu.VMEM_SHARED((NND, WROW), jnp.float32)],
  )


def _edge_math(rowsd, rowss, ea, we, b1, wc, w2, b2, w4, b4, w5, b5):
  g = rowsd[:, :HID] + rowss[:, :HID]
  rel = rowsd[:, HID:HID + 3] - rowss[:, HID:HID + 3]
  rel_d = jnp.sum(rel * rel, axis=1, keepdims=True)
  x1 = _silu(g + jnp.dot(ea, we, preferred_element_type=jnp.float32)
             + rel_d * wc + b1)
  m = _silu(jnp.dot(x1, w2, preferred_element_type=jnp.float32) + b2)
  c1 = _silu(jnp.dot(m, w4, preferred_element_type=jnp.float32) + b4)
  cw = jnp.dot(c1, w5, preferred_element_type=jnp.float32) + b5
  pay = jnp.concatenate(
      [m, rel * cw, jnp.zeros((m.shape[0], WROW - HID - 3), jnp.float32)],
      axis=1)
  return m, pay


def _edge_body_mid(rowsd_r, rowss_r, ea_r, we_r, b1_r, wc_r, w2_r, b2_r,
                   w3_r, b3_r, w4_r, b4_r, w5_r, b5_r, eaout_r, pay_r):
  ea = ea_r[...]
  m, pay = _edge_math(rowsd_r[...], rowss_r[...], ea, we_r[...], b1_r[...],
                      wc_r[...], w2_r[...], b2_r[...], w4_r[...], b4_r[...],
                      w5_r[...], b5_r[...])
  eaout_r[...] = jnp.dot(m, w3_r[...], preferred_element_type=jnp.float32) \
      + b3_r[...] + ea
  pay_r[...] = pay


def _edge_body_last(rowsd_r, rowss_r, ea_r, we_r, b1_r, wc_r, w2_r, b2_r,
                    w4_r, b4_r, w5_r, b5_r, pay_r):
  _, pay = _edge_math(rowsd_r[...], rowss_r[...], ea_r[...], we_r[...],
                      b1_r[...], wc_r[...], w2_r[...], b2_r[...], w4_r[...],
                      b4_r[...], w5_r[...], b5_r[...])
  pay_r[...] = pay


def _full_spec(arr):
  nd = len(arr.shape)
  return pl.BlockSpec(arr.shape, lambda i, _n=nd: (0,) * _n)


def _edge_specs(ws):
  row_spec = pl.BlockSpec((ET, WROW), lambda i: (i, 0))
  ea_spec = pl.BlockSpec((ET, HID), lambda i: (i, 0))
  w_specs = [_full_spec(w) for w in ws]
  return row_spec, ea_spec, w_specs


def _call_edge_mid(rowsd, rowss, ea, we, b1, wc, w2, b2, w3, b3, w4, b4,
                   w5, b5):
  row_spec, ea_spec, w_specs = _edge_specs(
      [we, b1, wc, w2, b2, w3, b3, w4, b4, w5, b5])
  return pl.pallas_call(
      _edge_body_mid,
      grid=(NED // ET,),
      in_specs=[row_spec, row_spec, ea_spec] + w_specs,
      out_specs=[ea_spec, pl.BlockSpec((ET, WROW), lambda i: (i, 0))],
      out_shape=[jax.ShapeDtypeStruct((NED, HID), jnp.float32),
                 jax.ShapeDtypeStruct((NED, WROW), jnp.float32)],
  )(rowsd, rowss, ea, we, b1, wc, w2, b2, w3, b3, w4, b4, w5, b5)


def _call_edge_last(rowsd, rowss, ea, we, b1, wc, w2, b2, w4, b4, w5, b5):
  row_spec, ea_spec, w_specs = _edge_specs(
      [we, b1, wc, w2, b2, w4, b4, w5, b5])
  return pl.pallas_call(
      _edge_body_last,
      grid=(NED // ET,),
      in_specs=[row_spec, row_spec, ea_spec] + w_specs,
      out_specs=[pl.BlockSpec((ET, WROW), lambda i: (i, 0))],
      out_shape=[jax.ShapeDtypeStruct((NED, WROW), jnp.float32)],
  )(rowsd, rowss, ea, we, b1, wc, w2, b2, w4, b4, w5, b5)[0]


def _node_math(feats, acc, td, scale, shift, wn1, bn1, wn2, bn2, g_ln, b_ln,
               wf1, bf1, wf2, bf2):
  m_i = acc[0, :, :HID] + acc[1, :, :HID]
  delta = acc[0, :, HID:HID + 3] + acc[1, :, HID:HID + 3]
  pos = td[:, HID:HID + 3] + delta
  cat = jnp.concatenate([feats, m_i], axis=1)
  nh = jnp.dot(_silu(jnp.dot(cat, wn1, preferred_element_type=jnp.float32)
                     + bn1), wn2, preferred_element_type=jnp.float32) \
      + bn2 + feats
  f = nh * (scale + 1.0) + shift
  denom = float(NND * HID)
  mean = jnp.sum(f) / denom
  xc = f - mean
  var = jnp.sum(xc * xc) / denom
  fn = xc * lax.rsqrt(var + 1e-5) * g_ln + b_ln
  u = jnp.dot(fn, wf1, preferred_element_type=jnp.float32) + bf1
  gl = 0.5 * u * (1.0 + lax.erf(u * (1.0 / math.sqrt(2.0))))
  f2 = jnp.dot(gl, wf2, preferred_element_type=jnp.float32) + bf2 + fn
  return f2, pos


def _node_body_mid(feats_r, acc_r, td_r, scale_r, shift_r, wn1_r, bn1_r,
                   wn2_r, bn2_r, g_r, b_r, wf1_r, bf1_r, wf2_r, bf2_r,
                   wdn_r, wsn_r, featso_r, tdo_r, tso_r):
  f2, pos = _node_math(feats_r[...], acc_r[...], td_r[...], scale_r[...],
                       shift_r[...], wn1_r[...], bn1_r[...], wn2_r[...],
                       bn2_r[...], g_r[...], b_r[...], wf1_r[...], bf1_r[...],
                       wf2_r[...], bf2_r[...])
  featso_r[...] = f2
  pad = jnp.zeros((NND, WROW - HID - 3), jnp.float32)
  tdo_r[...] = jnp.concatenate(
      [jnp.dot(f2, wdn_r[...], preferred_element_type=jnp.float32), pos, pad],
      axis=1)
  tso_r[...] = jnp.concatenate(
      [jnp.dot(f2, wsn_r[...], preferred_element_type=jnp.float32), pos, pad],
      axis=1)


def _node_body_last(feats_r, acc_r, td_r, scale_r, shift_r, wn1_r, bn1_r,
                    wn2_r, bn2_r, g_r, b_r, wf1_r, bf1_r, wf2_r, bf2_r,
                    wlin_r, blin_r, out_r):
  f2, _ = _node_math(feats_r[...], acc_r[...], td_r[...], scale_r[...],
                     shift_r[...], wn1_r[...], bn1_r[...], wn2_r[...],
                     bn2_r[...], g_r[...], b_r[...], wf1_r[...], bf1_r[...],
                     wf2_r[...], bf2_r[...])
  out_r[...] = jnp.dot(f2, wlin_r[...],
                       preferred_element_type=jnp.float32) + blin_r[...]


def _call_node_mid(feats, acc, td, scale, shift, wn1, bn1, wn2, bn2, g_ln,
                   b_ln, wf1, bf1, wf2, bf2, wdn, wsn):
  return pl.pallas_call(
      _node_body_mid,
      out_shape=[jax.ShapeDtypeStruct((NND, HID), jnp.float32),
                 jax.ShapeDtypeStruct((NND, WROW), jnp.float32),
                 jax.ShapeDtypeStruct((NND, WROW), jnp.float32)],
  )(feats, acc, td, scale, shift, wn1, bn1, wn2, bn2, g_ln, b_ln, wf1, bf1,
    wf2, bf2, wdn, wsn)


def _call_node_last(feats, acc, td, scale, shift, wn1, bn1, wn2, bn2, g_ln,
                    b_ln, wf1, bf1, wf2, bf2, wlin, blin):
  return pl.pallas_call(
      _node_body_last,
      out_shape=jax.ShapeDtypeStruct((NND, ODIM), jnp.float32),
  )(feats, acc, td, scale, shift, wn1, bn1, wn2, bn2, g_ln, b_ln, wf1, bf1,
    wf2, bf2, wlin, blin)


def _pro_body(x_r, ex_r, pos_r, time_r, wt0_r, bt0_r, wt1_r, bt1_r, wtl_r,
              btl_r, wd0_r, ws0_r, featso_r, tdo_r, tso_r, tembo_r):
  half = HID // 2
  k = math.log(10000.0) / (half - 1)
  f = jnp.exp(lax.broadcasted_iota(jnp.int32, (1, half), 1).astype(jnp.float32)
              * (-k))
  e = time_r[...] * f
  emb = jnp.concatenate([jnp.sin(e), jnp.cos(e)], axis=1)
  t1 = _silu(jnp.dot(emb, wt0_r[...], preferred_element_type=jnp.float32)
             + bt0_r[...])
  t = jnp.dot(t1, wt1_r[...], preferred_element_type=jnp.float32) + bt1_r[...]
  st = _silu(t)
  tembo_r[...] = jnp.concatenate(
      [jnp.dot(st, wtl_r[l], preferred_element_type=jnp.float32) + btl_r[l]
       for l in range(NL)], axis=0)
  feats = jnp.concatenate([x_r[...], ex_r[...]], axis=1)
  featso_r[...] = feats
  pos = pos_r[...]
  pad = jnp.zeros((NND, WROW - HID - 3), jnp.float32)
  tdo_r[...] = jnp.concatenate(
      [jnp.dot(feats, wd0_r[...], preferred_element_type=jnp.float32), pos,
       pad], axis=1)
  tso_r[...] = jnp.concatenate(
      [jnp.dot(feats, ws0_r[...], preferred_element_type=jnp.float32), pos,
       pad], axis=1)


def _call_prologue(x, extra_x, pos, time, wt0, bt0, wt1, bt1, wtl, btl,
                   wd0, ws0):
  return pl.pallas_call(
      _pro_body,
      out_shape=[jax.ShapeDtypeStruct((NND, HID), jnp.float32),
                 jax.ShapeDtypeStruct((NND, WROW), jnp.float32),
                 jax.ShapeDtypeStruct((NND, WROW), jnp.float32),
                 jax.ShapeDtypeStruct((NL, 2 * HID), jnp.float32)],
  )(x, extra_x, pos, time, wt0, bt0, wt1, bt1, wtl, btl, wd0, ws0)


def kernel(x, pos, extra_x, edge_attr, ss, time, params, edge_index, batch):
  del ss, batch
  layers = params["layers"]

  src3 = edge_index[0].reshape(NW, NCHUNK, CH)
  dst3 = edge_index[1].reshape(NW, NCHUNK, CH)
  zeros_stripe = jnp.zeros((STRIPE_LAST, WROW), jnp.float32)

  wtl = jnp.stack([l["time"]["w"] for l in layers])
  btl = jnp.stack([l["time"]["b"][None, :] for l in layers])

  def w1_slices(l):
    w1 = layers[l]["edge_mlp"][0]["w"]
    return (w1[0:HID], w1[HID:2 * HID], w1[2 * HID:2 * HID + 1],
            w1[2 * HID + 1:])

  wd0, ws0, _, _ = w1_slices(0)
  tm = params["time_mlp"]
  feats, td, ts, temb = _call_prologue(
      x, extra_x, pos, time.reshape(1, 1), tm[0]["w"], tm[0]["b"][None, :],
      tm[1]["w"], tm[1]["b"][None, :], wtl, btl, wd0, ws0)

  ea = edge_attr
  out = None
  for l in range(NL):
    lay = layers[l]
    _, _, wc, we = w1_slices(l)
    b1 = lay["edge_mlp"][0]["b"][None, :]
    w2, b2 = lay["edge_mlp"][1]["w"], lay["edge_mlp"][1]["b"][None, :]
    w3, b3 = lay["edge_upd"]["w"], lay["edge_upd"]["b"][None, :]
    w4, b4 = lay["coors_mlp"][0]["w"], lay["coors_mlp"][0]["b"][None, :]
    w5, b5 = lay["coors_mlp"][1]["w"], lay["coors_mlp"][1]["b"][None, :]

    rowsd, rowss = _sc_gather()(td, ts, dst3, src3)
    if l < NL - 1:
      ea, payload = _call_edge_mid(rowsd, rowss, ea, we, b1, wc, w2, b2, w3,
                                   b3, w4, b4, w5, b5)
    else:
      payload = _call_edge_last(rowsd, rowss, ea, we, b1, wc, w2, b2, w4, b4,
                                w5, b5)
    acc = _sc_scatter()(payload, dst3, zeros_stripe)

    scale = temb[l:l + 1, :HID]
    shift = temb[l:l + 1, HID:]
    wn1 = lay["node_mlp"][0]["w"]
    bn1 = lay["node_mlp"][0]["b"][None, :]
    wn2 = lay["node_mlp"][1]["w"]
    bn2 = lay["node_mlp"][1]["b"][None, :]
    g_ln = lay["ff_norm"]["g"][None, :]
    b_ln = lay["ff_norm"]["be"][None, :]
    wf1 = lay["ff"][0]["w"]
    bf1 = lay["ff"][0]["b"][None, :]
    wf2 = lay["ff"][1]["w"]
    bf2 = lay["ff"][1]["b"][None, :]
    if l < NL - 1:
      wdn, wsn, _, _ = w1_slices(l + 1)
      feats, td, ts = _call_node_mid(feats, acc, td, scale, shift, wn1, bn1,
                                     wn2, bn2, g_ln, b_ln, wf1, bf1, wf2, bf2,
                                     wdn, wsn)
    else:
      out = _call_node_last(feats, acc, td, scale, shift, wn1, bn1, wn2, bn2,
                            g_ln, b_ln, wf1, bf1, wf2, bf2, params["lin"]["w"],
                            params["lin"]["b"][None, :])
  return out


# speedup vs baseline: 4.5520x; 1.0109x over previous
import functools
import math

import jax
import jax.numpy as jnp
from jax import lax
from jax.experimental import pallas as pl
from jax.experimental.pallas import tpu as pltpu
from jax.experimental.pallas import tpu_sc as plsc

NND = 10000
NED = 320000
HID = 64
NL = 4
ODIM = 20
WROW = 128
PW = 128
NC, NS = 2, 16
NW = NC * NS
EPT = NED // NW
CH = 80
NCHUNK = EPT // CH
STRIPE = 624
STRIPE_LAST = NND - (NS - 1) * STRIPE
ET = 2000

@functools.cache
def _sc_mesh():
  return plsc.VectorSubcoreMesh(
      core_axis_name="c", subcore_axis_name="s", num_cores=NC, num_subcores=NS)


def _silu(x):
  return x * jax.nn.sigmoid(x)


def _sc_gather_body(td_hbm, ts_hbm, dst_hbm, src_hbm, outd_hbm, outs_hbm,
                    idxd_v, idxs_v, bufd0, bufd1, bufs0, bufs1,
                    gsd0, gsd1, gss0, gss1, wsd0, wsd1, wss0, wss1):
  cid = lax.axis_index("c")
  sid = lax.axis_index("s")
  wid = cid * NS + sid
  pltpu.sync_copy(dst_hbm.at[wid], idxd_v)
  pltpu.sync_copy(src_hbm.at[wid], idxs_v)
  base = wid * EPT
  bufd = (bufd0, bufd1)
  bufs = (bufs0, bufs1)
  gsd = (gsd0, gsd1)
  gss = (gss0, gss1)
  wsd = (wsd0, wsd1)
  wss = (wss0, wss1)

  def wr_desc(b):
    return (pltpu.make_async_copy(bufd[b], outd_hbm.at[pl.ds(base, CH)],
                                  wsd[b]),
            pltpu.make_async_copy(bufs[b], outs_hbm.at[pl.ds(base, CH)],
                                  wss[b]))

  def gathers(j0):
    gd = []
    for b in range(2):
      j = j0 + b
      gd.append(pltpu.async_copy(td_hbm.at[idxd_v.at[j]], bufd[b], gsd[b]))
      gd.append(pltpu.async_copy(ts_hbm.at[idxs_v.at[j]], bufs[b], gss[b]))
    return gd

  def writes(j0, gd):
    for b in range(2):
      j = j0 + b
      row0 = base + j * CH
      gd[2 * b].wait()
      pltpu.async_copy(bufd[b], outd_hbm.at[pl.ds(row0, CH)], wsd[b])
      gd[2 * b + 1].wait()
      pltpu.async_copy(bufs[b], outs_hbm.at[pl.ds(row0, CH)], wss[b])

  writes(0, gathers(0))

  @pl.loop(2, NCHUNK - 1, step=2)
  def _(j0):
    gd = []
    for b in range(2):
      j = j0 + b
      dd, ds_ = wr_desc(b)
      dd.wait()
      ds_.wait()
      gd.append(pltpu.async_copy(td_hbm.at[idxd_v.at[j]], bufd[b], gsd[b]))
      gd.append(pltpu.async_copy(ts_hbm.at[idxs_v.at[j]], bufs[b], gss[b]))
    writes(j0, gd)

  j = NCHUNK - 1
  for b in range(2):
    dd, ds_ = wr_desc(b)
    dd.wait()
    ds_.wait()
  g0 = pltpu.async_copy(td_hbm.at[idxd_v.at[j]], bufd[0], gsd[0])
  g1 = pltpu.async_copy(ts_hbm.at[idxs_v.at[j]], bufs[0], gss[0])
  row0 = base + j * CH
  g0.wait()
  pltpu.async_copy(bufd[0], outd_hbm.at[pl.ds(row0, CH)], wsd[0]).wait()
  g1.wait()
  pltpu.async_copy(bufs[0], outs_hbm.at[pl.ds(row0, CH)], wss[0]).wait()


@functools.cache
def _sc_gather():
  return pl.kernel(
      _sc_gather_body,
      out_type=[jax.ShapeDtypeStruct((NED, WROW), jnp.float32),
                jax.ShapeDtypeStruct((NED, WROW), jnp.float32)],
      mesh=_sc_mesh(),
      scratch_types=[pltpu.VMEM((NCHUNK, CH), jnp.int32),
                     pltpu.VMEM((NCHUNK, CH), jnp.int32)]
                    + [pltpu.VMEM((CH, WROW), jnp.float32)] * 4
                    + [pltpu.SemaphoreType.DMA] * 8,
  )


def _sc_scatter_body(pay_hbm, dst_hbm, zeros_hbm, out_hbm,
                     idx_v, pbuf0, pbuf1, sem0, sem1, acc_sh):
  cid = lax.axis_index("c")
  sid = lax.axis_index("s")
  wid = cid * NS + sid
  pltpu.sync_copy(dst_hbm.at[wid], idx_v)

  @pl.when(sid < NS - 1)
  def _():
    pltpu.sync_copy(zeros_hbm.at[pl.ds(0, STRIPE)],
                    acc_sh.at[pl.ds(sid * STRIPE, STRIPE)])

  @pl.when(sid == NS - 1)
  def _():
    pltpu.sync_copy(zeros_hbm, acc_sh.at[pl.ds((NS - 1) * STRIPE,
                                               STRIPE_LAST)])

  plsc.subcore_barrier()
  pbuf = (pbuf0, pbuf1)
  sem = (sem0, sem1)
  base = wid * EPT

  def pair(j0, nb):
    ld = []
    for b in range(nb):
      j = j0 + b
      ld.append(pltpu.async_copy(pay_hbm.at[pl.ds(base + j * CH, CH)],
                                 pbuf[b], sem[b]))
    for b in range(nb):
      ld[b].wait()
      pltpu.sync_copy(pbuf[b], acc_sh.at[idx_v.at[j0 + b]], add=True)

  @pl.loop(0, NCHUNK - 1, step=2)
  def _(j0):
    pair(j0, 2)

  pair(NCHUNK - 1, 1)

  plsc.subcore_barrier()

  @pl.when(sid < NS - 1)
  def _():
    pltpu.sync_copy(
        acc_sh.at[pl.ds(sid * STRIPE, STRIPE)],
        out_hbm.at[cid, pl.ds(sid * STRIPE, STRIPE)])

  @pl.when(sid == NS - 1)
  def _():
    pltpu.sync_copy(
        acc_sh.at[pl.ds((NS - 1) * STRIPE, STRIPE_LAST)],
        out_hbm.at[cid, pl.ds((NS - 1) * STRIPE, STRIPE_LAST)])


@functools.cache
def _sc_scatter():
  return pl.kernel(
      _sc_scatter_body,
      out_type=jax.ShapeDtypeStruct((NC, NND, PW), jnp.float32),
      mesh=_sc_mesh(),
      scratch_types=[pltpu.VMEM((NCHUNK, CH), jnp.int32)]
                    + [pltpu.VMEM((CH, PW), jnp.float32)] * 2
                    + [pltpu.SemaphoreType.DMA] * 2
                    + [pltpu.VMEM_SHARED((NND, PW), jnp.float32)],
  )


def _edge_math(rowsd, rowss, ea, we, b1, wc, w2, b2, w4, b4, w5, b5):
  g = rowsd[:, :HID] + rowss[:, :HID]
  rel = rowsd[:, HID:HID + 3] - rowss[:, HID:HID + 3]
  rel_d = jnp.sum(rel * rel, axis=1, keepdims=True)
  x1 = _silu(g + jnp.dot(ea, we, preferred_element_type=jnp.float32)
             + rel_d * wc + b1)
  m = _silu(jnp.dot(x1, w2, preferred_element_type=jnp.float32) + b2)
  c1 = _silu(jnp.dot(m, w4, preferred_element_type=jnp.float32) + b4)
  cw = jnp.dot(c1, w5, preferred_element_type=jnp.float32) + b5
  pay = jnp.concatenate(
      [m, rel * cw, jnp.zeros((m.shape[0], PW - HID - 3), jnp.float32)],
      axis=1)
  return m, pay


def _edge_body_mid(rowsd_r, rowss_r, ea_r, we_r, b1_r, wc_r, w2_r, b2_r,
                   w3_r, b3_r, w4_r, b4_r, w5_r, b5_r, eaout_r, pay_r):
  ea = ea_r[...]
  m, pay = _edge_math(rowsd_r[...], rowss_r[...], ea, we_r[...], b1_r[...],
                      wc_r[...], w2_r[...], b2_r[...], w4_r[...], b4_r[...],
                      w5_r[...], b5_r[...])
  eaout_r[...] = jnp.dot(m, w3_r[...], preferred_element_type=jnp.float32) \
      + b3_r[...] + ea
  pay_r[...] = pay


def _edge_body_last(rowsd_r, rowss_r, ea_r, we_r, b1_r, wc_r, w2_r, b2_r,
                    w4_r, b4_r, w5_r, b5_r, pay_r):
  _, pay = _edge_math(rowsd_r[...], rowss_r[...], ea_r[...], we_r[...],
                      b1_r[...], wc_r[...], w2_r[...], b2_r[...], w4_r[...],
                      b4_r[...], w5_r[...], b5_r[...])
  pay_r[...] = pay


def _full_spec(arr):
  nd = len(arr.shape)
  return pl.BlockSpec(arr.shape, lambda i, _n=nd: (0,) * _n)


def _edge_specs(ws):
  row_spec = pl.BlockSpec((ET, WROW), lambda i: (i, 0))
  ea_spec = pl.BlockSpec((ET, HID), lambda i: (i, 0))
  w_specs = [_full_spec(w) for w in ws]
  return row_spec, ea_spec, w_specs


def _call_edge_mid(rowsd, rowss, ea, we, b1, wc, w2, b2, w3, b3, w4, b4,
                   w5, b5):
  row_spec, ea_spec, w_specs = _edge_specs(
      [we, b1, wc, w2, b2, w3, b3, w4, b4, w5, b5])
  return pl.pallas_call(
      _edge_body_mid,
      grid=(NED // ET,),
      in_specs=[row_spec, row_spec, ea_spec] + w_specs,
      out_specs=[ea_spec, pl.BlockSpec((ET, PW), lambda i: (i, 0))],
      out_shape=[jax.ShapeDtypeStruct((NED, HID), jnp.float32),
                 jax.ShapeDtypeStruct((NED, PW), jnp.float32)],
  )(rowsd, rowss, ea, we, b1, wc, w2, b2, w3, b3, w4, b4, w5, b5)


def _call_edge_last(rowsd, rowss, ea, we, b1, wc, w2, b2, w4, b4, w5, b5):
  row_spec, ea_spec, w_specs = _edge_specs(
      [we, b1, wc, w2, b2, w4, b4, w5, b5])
  return pl.pallas_call(
      _edge_body_last,
      grid=(NED // ET,),
      in_specs=[row_spec, row_spec, ea_spec] + w_specs,
      out_specs=[pl.BlockSpec((ET, PW), lambda i: (i, 0))],
      out_shape=[jax.ShapeDtypeStruct((NED, PW), jnp.float32)],
  )(rowsd, rowss, ea, we, b1, wc, w2, b2, w4, b4, w5, b5)[0]


def _node_math(feats, acc, td, scale, shift, wn1, bn1, wn2, bn2, g_ln, b_ln,
               wf1, bf1, wf2, bf2):
  m_i = acc[0, :, :HID] + acc[1, :, :HID]
  delta = acc[0, :, HID:HID + 3] + acc[1, :, HID:HID + 3]
  pos = td[:, HID:HID + 3] + delta
  cat = jnp.concatenate([feats, m_i], axis=1)
  nh = jnp.dot(_silu(jnp.dot(cat, wn1, preferred_element_type=jnp.float32)
                     + bn1), wn2, preferred_element_type=jnp.float32) \
      + bn2 + feats
  f = nh * (scale + 1.0) + shift
  denom = float(NND * HID)
  mean = jnp.sum(f) / denom
  xc = f - mean
  var = jnp.sum(xc * xc) / denom
  fn = xc * lax.rsqrt(var + 1e-5) * g_ln + b_ln
  u = jnp.dot(fn, wf1, preferred_element_type=jnp.float32) + bf1
  gl = 0.5 * u * (1.0 + lax.erf(u * (1.0 / math.sqrt(2.0))))
  f2 = jnp.dot(gl, wf2, preferred_element_type=jnp.float32) + bf2 + fn
  return f2, pos


def _node_body_mid(feats_r, acc_r, td_r, scale_r, shift_r, wn1_r, bn1_r,
                   wn2_r, bn2_r, g_r, b_r, wf1_r, bf1_r, wf2_r, bf2_r,
                   wdn_r, wsn_r, featso_r, tdo_r, tso_r):
  f2, pos = _node_math(feats_r[...], acc_r[...], td_r[...], scale_r[...],
                       shift_r[...], wn1_r[...], bn1_r[...], wn2_r[...],
                       bn2_r[...], g_r[...], b_r[...], wf1_r[...], bf1_r[...],
                       wf2_r[...], bf2_r[...])
  featso_r[...] = f2
  pad = jnp.zeros((NND, WROW - HID - 3), jnp.float32)
  tdo_r[...] = jnp.concatenate(
      [jnp.dot(f2, wdn_r[...], preferred_element_type=jnp.float32), pos, pad],
      axis=1)
  tso_r[...] = jnp.concatenate(
      [jnp.dot(f2, wsn_r[...], preferred_element_type=jnp.float32), pos, pad],
      axis=1)


def _node_body_last(feats_r, acc_r, td_r, scale_r, shift_r, wn1_r, bn1_r,
                    wn2_r, bn2_r, g_r, b_r, wf1_r, bf1_r, wf2_r, bf2_r,
                    wlin_r, blin_r, out_r):
  f2, _ = _node_math(feats_r[...], acc_r[...], td_r[...], scale_r[...],
                     shift_r[...], wn1_r[...], bn1_r[...], wn2_r[...],
                     bn2_r[...], g_r[...], b_r[...], wf1_r[...], bf1_r[...],
                     wf2_r[...], bf2_r[...])
  out_r[...] = jnp.dot(f2, wlin_r[...],
                       preferred_element_type=jnp.float32) + blin_r[...]


def _call_node_mid(feats, acc, td, scale, shift, wn1, bn1, wn2, bn2, g_ln,
                   b_ln, wf1, bf1, wf2, bf2, wdn, wsn):
  return pl.pallas_call(
      _node_body_mid,
      out_shape=[jax.ShapeDtypeStruct((NND, HID), jnp.float32),
                 jax.ShapeDtypeStruct((NND, WROW), jnp.float32),
                 jax.ShapeDtypeStruct((NND, WROW), jnp.float32)],
  )(feats, acc, td, scale, shift, wn1, bn1, wn2, bn2, g_ln, b_ln, wf1, bf1,
    wf2, bf2, wdn, wsn)


def _call_node_last(feats, acc, td, scale, shift, wn1, bn1, wn2, bn2, g_ln,
                    b_ln, wf1, bf1, wf2, bf2, wlin, blin):
  return pl.pallas_call(
      _node_body_last,
      out_shape=jax.ShapeDtypeStruct((NND, ODIM), jnp.float32),
  )(feats, acc, td, scale, shift, wn1, bn1, wn2, bn2, g_ln, b_ln, wf1, bf1,
    wf2, bf2, wlin, blin)


def _pro_body(x_r, ex_r, pos_r, time_r, wt0_r, bt0_r, wt1_r, bt1_r, wtl_r,
              btl_r, wd0_r, ws0_r, featso_r, tdo_r, tso_r, tembo_r):
  half = HID // 2
  k = math.log(10000.0) / (half - 1)
  f = jnp.exp(lax.broadcasted_iota(jnp.int32, (1, half), 1).astype(jnp.float32)
              * (-k))
  e = time_r[...] * f
  emb = jnp.concatenate([jnp.sin(e), jnp.cos(e)], axis=1)
  t1 = _silu(jnp.dot(emb, wt0_r[...], preferred_element_type=jnp.float32)
             + bt0_r[...])
  t = jnp.dot(t1, wt1_r[...], preferred_element_type=jnp.float32) + bt1_r[...]
  st = _silu(t)
  tembo_r[...] = jnp.concatenate(
      [jnp.dot(st, wtl_r[l], preferred_element_type=jnp.float32) + btl_r[l]
       for l in range(NL)], axis=0)
  feats = jnp.concatenate([x_r[...], ex_r[...]], axis=1)
  featso_r[...] = feats
  pos = pos_r[...]
  pad = jnp.zeros((NND, WROW - HID - 3), jnp.float32)
  tdo_r[...] = jnp.concatenate(
      [jnp.dot(feats, wd0_r[...], preferred_element_type=jnp.float32), pos,
       pad], axis=1)
  tso_r[...] = jnp.concatenate(
      [jnp.dot(feats, ws0_r[...], preferred_element_type=jnp.float32), pos,
       pad], axis=1)


def _call_prologue(x, extra_x, pos, time, wt0, bt0, wt1, bt1, wtl, btl,
                   wd0, ws0):
  return pl.pallas_call(
      _pro_body,
      out_shape=[jax.ShapeDtypeStruct((NND, HID), jnp.float32),
                 jax.ShapeDtypeStruct((NND, WROW), jnp.float32),
                 jax.ShapeDtypeStruct((NND, WROW), jnp.float32),
                 jax.ShapeDtypeStruct((NL, 2 * HID), jnp.float32)],
  )(x, extra_x, pos, time, wt0, bt0, wt1, bt1, wtl, btl, wd0, ws0)


def kernel(x, pos, extra_x, edge_attr, ss, time, params, edge_index, batch):
  del ss, batch
  layers = params["layers"]

  src3 = edge_index[0].reshape(NW, NCHUNK, CH)
  dst3 = edge_index[1].reshape(NW, NCHUNK, CH)
  zeros_stripe = jnp.zeros((STRIPE_LAST, PW), jnp.float32)

  wtl = jnp.stack([l["time"]["w"] for l in layers])
  btl = jnp.stack([l["time"]["b"][None, :] for l in layers])

  def w1_slices(l):
    w1 = layers[l]["edge_mlp"][0]["w"]
    return (w1[0:HID], w1[HID:2 * HID], w1[2 * HID:2 * HID + 1],
            w1[2 * HID + 1:])

  wd0, ws0, _, _ = w1_slices(0)
  tm = params["time_mlp"]
  feats, td, ts, temb = _call_prologue(
      x, extra_x, pos, time.reshape(1, 1), tm[0]["w"], tm[0]["b"][None, :],
      tm[1]["w"], tm[1]["b"][None, :], wtl, btl, wd0, ws0)

  ea = edge_attr
  out = None
  for l in range(NL):
    lay = layers[l]
    _, _, wc, we = w1_slices(l)
    b1 = lay["edge_mlp"][0]["b"][None, :]
    w2, b2 = lay["edge_mlp"][1]["w"], lay["edge_mlp"][1]["b"][None, :]
    w3, b3 = lay["edge_upd"]["w"], lay["edge_upd"]["b"][None, :]
    w4, b4 = lay["coors_mlp"][0]["w"], lay["coors_mlp"][0]["b"][None, :]
    w5, b5 = lay["coors_mlp"][1]["w"], lay["coors_mlp"][1]["b"][None, :]

    rowsd, rowss = _sc_gather()(td, ts, dst3, src3)
    if l < NL - 1:
      ea, payload = _call_edge_mid(rowsd, rowss, ea, we, b1, wc, w2, b2, w3,
                                   b3, w4, b4, w5, b5)
    else:
      payload = _call_edge_last(rowsd, rowss, ea, we, b1, wc, w2, b2, w4, b4,
                                w5, b5)
    acc = _sc_scatter()(payload, dst3, zeros_stripe)

    scale = temb[l:l + 1, :HID]
    shift = temb[l:l + 1, HID:]
    wn1 = lay["node_mlp"][0]["w"]
    bn1 = lay["node_mlp"][0]["b"][None, :]
    wn2 = lay["node_mlp"][1]["w"]
    bn2 = lay["node_mlp"][1]["b"][None, :]
    g_ln = lay["ff_norm"]["g"][None, :]
    b_ln = lay["ff_norm"]["be"][None, :]
    wf1 = lay["ff"][0]["w"]
    bf1 = lay["ff"][0]["b"][None, :]
    wf2 = lay["ff"][1]["w"]
    bf2 = lay["ff"][1]["b"][None, :]
    if l < NL - 1:
      wdn, wsn, _, _ = w1_slices(l + 1)
      feats, td, ts = _call_node_mid(feats, acc, td, scale, shift, wn1, bn1,
                                     wn2, bn2, g_ln, b_ln, wf1, bf1, wf2, bf2,
                                     wdn, wsn)
    else:
      out = _call_node_last(feats, acc, td, scale, shift, wn1, bn1, wn2, bn2,
                            g_ln, b_ln, wf1, bf1, wf2, bf2, params["lin"]["w"],
                            params["lin"]["b"][None, :])
  return out


# speedup vs baseline: 4.8481x; 1.0650x over previous
import functools
import math

import jax
import jax.numpy as jnp
from jax import lax
from jax.experimental import pallas as pl
from jax.experimental.pallas import tpu as pltpu
from jax.experimental.pallas import tpu_sc as plsc

NND = 10000
NED = 320000
HID = 64
NL = 4
ODIM = 20
WROW = 128
PW = 128
NC, NS = 2, 16
NW = NC * NS
NED_H = NED // 2
EPT_H = NED_H // NW
CH = 40
NCHUNK = EPT_H // CH
STRIPE = 624
STRIPE_LAST = NND - (NS - 1) * STRIPE
ET = 2000

@functools.cache
def _sc_mesh():
  return plsc.VectorSubcoreMesh(
      core_axis_name="c", subcore_axis_name="s", num_cores=NC, num_subcores=NS)


def _silu(x):
  return x * jax.nn.sigmoid(x)


def _sc_gather_body(td_hbm, ts_hbm, dst_hbm, src_hbm, outd_hbm, outs_hbm,
                    idxd_v, idxs_v, bufd0, bufd1, bufs0, bufs1,
                    gsd0, gsd1, gss0, gss1, wsd0, wsd1, wss0, wss1):
  cid = lax.axis_index("c")
  sid = lax.axis_index("s")
  wid = cid * NS + sid
  pltpu.sync_copy(dst_hbm.at[wid], idxd_v)
  pltpu.sync_copy(src_hbm.at[wid], idxs_v)
  base = wid * EPT_H
  bufd = (bufd0, bufd1)
  bufs = (bufs0, bufs1)
  gsd = (gsd0, gsd1)
  gss = (gss0, gss1)
  wsd = (wsd0, wsd1)
  wss = (wss0, wss1)

  def wr_desc(b):
    return (pltpu.make_async_copy(bufd[b], outd_hbm.at[pl.ds(base, CH)],
                                  wsd[b]),
            pltpu.make_async_copy(bufs[b], outs_hbm.at[pl.ds(base, CH)],
                                  wss[b]))

  def gathers(j0):
    gd = []
    for b in range(2):
      j = j0 + b
      gd.append(pltpu.async_copy(td_hbm.at[idxd_v.at[j]], bufd[b], gsd[b]))
      gd.append(pltpu.async_copy(ts_hbm.at[idxs_v.at[j]], bufs[b], gss[b]))
    return gd

  def writes(j0, gd):
    for b in range(2):
      j = j0 + b
      row0 = base + j * CH
      gd[2 * b].wait()
      pltpu.async_copy(bufd[b], outd_hbm.at[pl.ds(row0, CH)], wsd[b])
      gd[2 * b + 1].wait()
      pltpu.async_copy(bufs[b], outs_hbm.at[pl.ds(row0, CH)], wss[b])

  writes(0, gathers(0))

  @pl.loop(2, NCHUNK - 1, step=2)
  def _(j0):
    gd = []
    for b in range(2):
      j = j0 + b
      dd, ds_ = wr_desc(b)
      dd.wait()
      ds_.wait()
      gd.append(pltpu.async_copy(td_hbm.at[idxd_v.at[j]], bufd[b], gsd[b]))
      gd.append(pltpu.async_copy(ts_hbm.at[idxs_v.at[j]], bufs[b], gss[b]))
    writes(j0, gd)

  j = NCHUNK - 1
  for b in range(2):
    dd, ds_ = wr_desc(b)
    dd.wait()
    ds_.wait()
  g0 = pltpu.async_copy(td_hbm.at[idxd_v.at[j]], bufd[0], gsd[0])
  g1 = pltpu.async_copy(ts_hbm.at[idxs_v.at[j]], bufs[0], gss[0])
  row0 = base + j * CH
  g0.wait()
  pltpu.async_copy(bufd[0], outd_hbm.at[pl.ds(row0, CH)], wsd[0]).wait()
  g1.wait()
  pltpu.async_copy(bufs[0], outs_hbm.at[pl.ds(row0, CH)], wss[0]).wait()


@functools.cache
def _sc_gather():
  return pl.kernel(
      _sc_gather_body,
      out_type=[jax.ShapeDtypeStruct((NED_H, WROW), jnp.float32),
                jax.ShapeDtypeStruct((NED_H, WROW), jnp.float32)],
      mesh=_sc_mesh(),
      scratch_types=[pltpu.VMEM((NCHUNK, CH), jnp.int32),
                     pltpu.VMEM((NCHUNK, CH), jnp.int32)]
                    + [pltpu.VMEM((CH, WROW), jnp.float32)] * 4
                    + [pltpu.SemaphoreType.DMA] * 8,
  )


def _sc_scatter_body(pay_hbm, dst_hbm, zeros_hbm, out_hbm,
                     idx_v, pbuf0, pbuf1, sem0, sem1, acc_sh):
  cid = lax.axis_index("c")
  sid = lax.axis_index("s")
  wid = cid * NS + sid
  pltpu.sync_copy(dst_hbm.at[wid], idx_v)

  @pl.when(sid < NS - 1)
  def _():
    pltpu.sync_copy(zeros_hbm.at[pl.ds(0, STRIPE)],
                    acc_sh.at[pl.ds(sid * STRIPE, STRIPE)])

  @pl.when(sid == NS - 1)
  def _():
    pltpu.sync_copy(zeros_hbm, acc_sh.at[pl.ds((NS - 1) * STRIPE,
                                               STRIPE_LAST)])

  plsc.subcore_barrier()
  pbuf = (pbuf0, pbuf1)
  sem = (sem0, sem1)
  base = wid * EPT_H

  def pair(j0, nb):
    ld = []
    for b in range(nb):
      j = j0 + b
      ld.append(pltpu.async_copy(pay_hbm.at[pl.ds(base + j * CH, CH)],
                                 pbuf[b], sem[b]))
    for b in range(nb):
      ld[b].wait()
      pltpu.sync_copy(pbuf[b], acc_sh.at[idx_v.at[j0 + b]], add=True)

  @pl.loop(0, NCHUNK - 1, step=2)
  def _(j0):
    pair(j0, 2)

  pair(NCHUNK - 1, 1)

  plsc.subcore_barrier()

  @pl.when(sid < NS - 1)
  def _():
    pltpu.sync_copy(
        acc_sh.at[pl.ds(sid * STRIPE, STRIPE)],
        out_hbm.at[cid, pl.ds(sid * STRIPE, STRIPE)])

  @pl.when(sid == NS - 1)
  def _():
    pltpu.sync_copy(
        acc_sh.at[pl.ds((NS - 1) * STRIPE, STRIPE_LAST)],
        out_hbm.at[cid, pl.ds((NS - 1) * STRIPE, STRIPE_LAST)])


@functools.cache
def _sc_scatter():
  return pl.kernel(
      _sc_scatter_body,
      out_type=jax.ShapeDtypeStruct((NC, NND, PW), jnp.float32),
      mesh=_sc_mesh(),
      scratch_types=[pltpu.VMEM((NCHUNK, CH), jnp.int32)]
                    + [pltpu.VMEM((CH, PW), jnp.float32)] * 2
                    + [pltpu.SemaphoreType.DMA] * 2
                    + [pltpu.VMEM_SHARED((NND, PW), jnp.float32)],
  )


def _edge_math(rowsd, rowss, ea, we, b1, wc, w2, b2, w4, b4, w5, b5):
  g = rowsd[:, :HID] + rowss[:, :HID]
  rel = rowsd[:, HID:HID + 3] - rowss[:, HID:HID + 3]
  rel_d = jnp.sum(rel * rel, axis=1, keepdims=True)
  x1 = _silu(g + jnp.dot(ea, we, preferred_element_type=jnp.float32)
             + rel_d * wc + b1)
  m = _silu(jnp.dot(x1, w2, preferred_element_type=jnp.float32) + b2)
  c1 = _silu(jnp.dot(m, w4, preferred_element_type=jnp.float32) + b4)
  cw = jnp.dot(c1, w5, preferred_element_type=jnp.float32) + b5
  pay = jnp.concatenate(
      [m, rel * cw, jnp.zeros((m.shape[0], PW - HID - 3), jnp.float32)],
      axis=1)
  return m, pay


def _edge_body_mid(rowsd_r, rowss_r, ea_r, we_r, b1_r, wc_r, w2_r, b2_r,
                   w3_r, b3_r, w4_r, b4_r, w5_r, b5_r, eaout_r, pay_r):
  ea = ea_r[...]
  m, pay = _edge_math(rowsd_r[...], rowss_r[...], ea, we_r[...], b1_r[...],
                      wc_r[...], w2_r[...], b2_r[...], w4_r[...], b4_r[...],
                      w5_r[...], b5_r[...])
  eaout_r[...] = jnp.dot(m, w3_r[...], preferred_element_type=jnp.float32) \
      + b3_r[...] + ea
  pay_r[...] = pay


def _edge_body_last(rowsd_r, rowss_r, ea_r, we_r, b1_r, wc_r, w2_r, b2_r,
                    w4_r, b4_r, w5_r, b5_r, pay_r):
  _, pay = _edge_math(rowsd_r[...], rowss_r[...], ea_r[...], we_r[...],
                      b1_r[...], wc_r[...], w2_r[...], b2_r[...], w4_r[...],
                      b4_r[...], w5_r[...], b5_r[...])
  pay_r[...] = pay


def _full_spec(arr):
  nd = len(arr.shape)
  return pl.BlockSpec(arr.shape, lambda i, _n=nd: (0,) * _n)


def _edge_specs(ws):
  row_spec = pl.BlockSpec((ET, WROW), lambda i: (i, 0))
  ea_spec = pl.BlockSpec((ET, HID), lambda i: (i, 0))
  w_specs = [_full_spec(w) for w in ws]
  return row_spec, ea_spec, w_specs


NSTEP_H = NED_H // ET


def _call_edge_mid(rowsd, rowss, ea, we, b1, wc, w2, b2, w3, b3, w4, b4,
                   w5, b5):
  row_spec, ea_spec, w_specs = _edge_specs(
      [we, b1, wc, w2, b2, w3, b3, w4, b4, w5, b5])
  return pl.pallas_call(
      _edge_body_mid,
      grid=(NSTEP_H,),
      in_specs=[row_spec, row_spec, ea_spec] + w_specs,
      out_specs=[ea_spec, pl.BlockSpec((ET, PW), lambda i: (i, 0))],
      out_shape=[jax.ShapeDtypeStruct((NED_H, HID), jnp.float32),
                 jax.ShapeDtypeStruct((NED_H, PW), jnp.float32)],
  )(rowsd, rowss, ea, we, b1, wc, w2, b2, w3, b3, w4, b4, w5, b5)


def _call_edge_last(rowsd, rowss, ea, we, b1, wc, w2, b2, w4, b4, w5, b5):
  row_spec, ea_spec, w_specs = _edge_specs(
      [we, b1, wc, w2, b2, w4, b4, w5, b5])
  return pl.pallas_call(
      _edge_body_last,
      grid=(NSTEP_H,),
      in_specs=[row_spec, row_spec, ea_spec] + w_specs,
      out_specs=[pl.BlockSpec((ET, PW), lambda i: (i, 0))],
      out_shape=[jax.ShapeDtypeStruct((NED_H, PW), jnp.float32)],
  )(rowsd, rowss, ea, we, b1, wc, w2, b2, w4, b4, w5, b5)[0]


def _node_math(feats, acca, accb, td, scale, shift, wn1, bn1, wn2, bn2, g_ln,
               b_ln, wf1, bf1, wf2, bf2):
  m_i = (acca[0, :, :HID] + acca[1, :, :HID]
         + accb[0, :, :HID] + accb[1, :, :HID])
  delta = (acca[0, :, HID:HID + 3] + acca[1, :, HID:HID + 3]
           + accb[0, :, HID:HID + 3] + accb[1, :, HID:HID + 3])
  pos = td[:, HID:HID + 3] + delta
  cat = jnp.concatenate([feats, m_i], axis=1)
  nh = jnp.dot(_silu(jnp.dot(cat, wn1, preferred_element_type=jnp.float32)
                     + bn1), wn2, preferred_element_type=jnp.float32) \
      + bn2 + feats
  f = nh * (scale + 1.0) + shift
  denom = float(NND * HID)
  mean = jnp.sum(f) / denom
  xc = f - mean
  var = jnp.sum(xc * xc) / denom
  fn = xc * lax.rsqrt(var + 1e-5) * g_ln + b_ln
  u = jnp.dot(fn, wf1, preferred_element_type=jnp.float32) + bf1
  gl = 0.5 * u * (1.0 + lax.erf(u * (1.0 / math.sqrt(2.0))))
  f2 = jnp.dot(gl, wf2, preferred_element_type=jnp.float32) + bf2 + fn
  return f2, pos


def _node_body_mid(feats_r, acca_r, accb_r, td_r, scale_r, shift_r, wn1_r,
                   bn1_r, wn2_r, bn2_r, g_r, b_r, wf1_r, bf1_r, wf2_r, bf2_r,
                   wdn_r, wsn_r, featso_r, tdo_r, tso_r):
  f2, pos = _node_math(feats_r[...], acca_r[...], accb_r[...], td_r[...],
                       scale_r[...], shift_r[...], wn1_r[...], bn1_r[...],
                       wn2_r[...], bn2_r[...], g_r[...], b_r[...], wf1_r[...],
                       bf1_r[...], wf2_r[...], bf2_r[...])
  featso_r[...] = f2
  pad = jnp.zeros((NND, WROW - HID - 3), jnp.float32)
  tdo_r[...] = jnp.concatenate(
      [jnp.dot(f2, wdn_r[...], preferred_element_type=jnp.float32), pos, pad],
      axis=1)
  tso_r[...] = jnp.concatenate(
      [jnp.dot(f2, wsn_r[...], preferred_element_type=jnp.float32), pos, pad],
      axis=1)


def _node_body_last(feats_r, acca_r, accb_r, td_r, scale_r, shift_r, wn1_r,
                    bn1_r, wn2_r, bn2_r, g_r, b_r, wf1_r, bf1_r, wf2_r, bf2_r,
                    wlin_r, blin_r, out_r):
  f2, _ = _node_math(feats_r[...], acca_r[...], accb_r[...], td_r[...],
                     scale_r[...], shift_r[...], wn1_r[...], bn1_r[...],
                     wn2_r[...], bn2_r[...], g_r[...], b_r[...], wf1_r[...],
                     bf1_r[...], wf2_r[...], bf2_r[...])
  out_r[...] = jnp.dot(f2, wlin_r[...],
                       preferred_element_type=jnp.float32) + blin_r[...]


def _call_node_mid(feats, acca, accb, td, scale, shift, wn1, bn1, wn2, bn2,
                   g_ln, b_ln, wf1, bf1, wf2, bf2, wdn, wsn):
  return pl.pallas_call(
      _node_body_mid,
      out_shape=[jax.ShapeDtypeStruct((NND, HID), jnp.float32),
                 jax.ShapeDtypeStruct((NND, WROW), jnp.float32),
                 jax.ShapeDtypeStruct((NND, WROW), jnp.float32)],
  )(feats, acca, accb, td, scale, shift, wn1, bn1, wn2, bn2, g_ln, b_ln, wf1,
    bf1, wf2, bf2, wdn, wsn)


def _call_node_last(feats, acca, accb, td, scale, shift, wn1, bn1, wn2, bn2,
                    g_ln, b_ln, wf1, bf1, wf2, bf2, wlin, blin):
  return pl.pallas_call(
      _node_body_last,
      out_shape=jax.ShapeDtypeStruct((NND, ODIM), jnp.float32),
  )(feats, acca, accb, td, scale, shift, wn1, bn1, wn2, bn2, g_ln, b_ln, wf1,
    bf1, wf2, bf2, wlin, blin)


def _pro_body(x_r, ex_r, pos_r, time_r, wt0_r, bt0_r, wt1_r, bt1_r, wtl_r,
              btl_r, wd0_r, ws0_r, featso_r, tdo_r, tso_r, tembo_r):
  half = HID // 2
  k = math.log(10000.0) / (half - 1)
  f = jnp.exp(lax.broadcasted_iota(jnp.int32, (1, half), 1).astype(jnp.float32)
              * (-k))
  e = time_r[...] * f
  emb = jnp.concatenate([jnp.sin(e), jnp.cos(e)], axis=1)
  t1 = _silu(jnp.dot(emb, wt0_r[...], preferred_element_type=jnp.float32)
             + bt0_r[...])
  t = jnp.dot(t1, wt1_r[...], preferred_element_type=jnp.float32) + bt1_r[...]
  st = _silu(t)
  tembo_r[...] = jnp.concatenate(
      [jnp.dot(st, wtl_r[l], preferred_element_type=jnp.float32) + btl_r[l]
       for l in range(NL)], axis=0)
  feats = jnp.concatenate([x_r[...], ex_r[...]], axis=1)
  featso_r[...] = feats
  pos = pos_r[...]
  pad = jnp.zeros((NND, WROW - HID - 3), jnp.float32)
  tdo_r[...] = jnp.concatenate(
      [jnp.dot(feats, wd0_r[...], preferred_element_type=jnp.float32), pos,
       pad], axis=1)
  tso_r[...] = jnp.concatenate(
      [jnp.dot(feats, ws0_r[...], preferred_element_type=jnp.float32), pos,
       pad], axis=1)


def _call_prologue(x, extra_x, pos, time, wt0, bt0, wt1, bt1, wtl, btl,
                   wd0, ws0):
  return pl.pallas_call(
      _pro_body,
      out_shape=[jax.ShapeDtypeStruct((NND, HID), jnp.float32),
                 jax.ShapeDtypeStruct((NND, WROW), jnp.float32),
                 jax.ShapeDtypeStruct((NND, WROW), jnp.float32),
                 jax.ShapeDtypeStruct((NL, 2 * HID), jnp.float32)],
  )(x, extra_x, pos, time, wt0, bt0, wt1, bt1, wtl, btl, wd0, ws0)


def kernel(x, pos, extra_x, edge_attr, ss, time, params, edge_index, batch):
  del ss, batch
  layers = params["layers"]

  src3 = [edge_index[0, h * NED_H:(h + 1) * NED_H].reshape(NW, NCHUNK, CH)
          for h in range(2)]
  dst3 = [edge_index[1, h * NED_H:(h + 1) * NED_H].reshape(NW, NCHUNK, CH)
          for h in range(2)]
  zeros_stripe = jnp.zeros((STRIPE_LAST, PW), jnp.float32)

  wtl = jnp.stack([l["time"]["w"] for l in layers])
  btl = jnp.stack([l["time"]["b"][None, :] for l in layers])

  def w1_slices(l):
    w1 = layers[l]["edge_mlp"][0]["w"]
    return (w1[0:HID], w1[HID:2 * HID], w1[2 * HID:2 * HID + 1],
            w1[2 * HID + 1:])

  wd0, ws0, _, _ = w1_slices(0)
  tm = params["time_mlp"]
  feats, td, ts, temb = _call_prologue(
      x, extra_x, pos, time.reshape(1, 1), tm[0]["w"], tm[0]["b"][None, :],
      tm[1]["w"], tm[1]["b"][None, :], wtl, btl, wd0, ws0)

  ea = [edge_attr[:NED_H], edge_attr[NED_H:]]
  out = None
  for l in range(NL):
    lay = layers[l]
    _, _, wc, we = w1_slices(l)
    b1 = lay["edge_mlp"][0]["b"][None, :]
    w2, b2 = lay["edge_mlp"][1]["w"], lay["edge_mlp"][1]["b"][None, :]
    w3, b3 = lay["edge_upd"]["w"], lay["edge_upd"]["b"][None, :]
    w4, b4 = lay["coors_mlp"][0]["w"], lay["coors_mlp"][0]["b"][None, :]
    w5, b5 = lay["coors_mlp"][1]["w"], lay["coors_mlp"][1]["b"][None, :]

    acc = [None, None]
    for h in range(2):
      rowsd, rowss = _sc_gather()(td, ts, dst3[h], src3[h])
      if l < NL - 1:
        ea[h], payload = _call_edge_mid(rowsd, rowss, ea[h], we, b1, wc, w2,
                                        b2, w3, b3, w4, b4, w5, b5)
      else:
        payload = _call_edge_last(rowsd, rowss, ea[h], we, b1, wc, w2, b2,
                                  w4, b4, w5, b5)
      acc[h] = _sc_scatter()(payload, dst3[h], zeros_stripe)

    scale = temb[l:l + 1, :HID]
    shift = temb[l:l + 1, HID:]
    wn1 = lay["node_mlp"][0]["w"]
    bn1 = lay["node_mlp"][0]["b"][None, :]
    wn2 = lay["node_mlp"][1]["w"]
    bn2 = lay["node_mlp"][1]["b"][None, :]
    g_ln = lay["ff_norm"]["g"][None, :]
    b_ln = lay["ff_norm"]["be"][None, :]
    wf1 = lay["ff"][0]["w"]
    bf1 = lay["ff"][0]["b"][None, :]
    wf2 = lay["ff"][1]["w"]
    bf2 = lay["ff"][1]["b"][None, :]
    if l < NL - 1:
      wdn, wsn, _, _ = w1_slices(l + 1)
      feats, td, ts = _call_node_mid(feats, acc[0], acc[1], td, scale, shift,
                                     wn1, bn1, wn2, bn2, g_ln, b_ln, wf1, bf1,
                                     wf2, bf2, wdn, wsn)
    else:
      out = _call_node_last(feats, acc[0], acc[1], td, scale, shift, wn1, bn1,
                            wn2, bn2, g_ln, b_ln, wf1, bf1, wf2, bf2,
                            params["lin"]["w"], params["lin"]["b"][None, :])
  return out
